# Initial kernel scaffold; baseline (speedup 1.0000x reference)
import jax
import jax.numpy as jnp
from jax.experimental import pallas as pl


def kernel(x, edge_index, W1, b1, W2, b2):
    raise NotImplementedError("write your pallas kernel here")


# speedup vs baseline: 14.4849x; 14.4849x over previous
import functools

import jax
import jax.numpy as jnp
from jax import lax
from jax.experimental import pallas as pl
from jax.experimental.pallas import tpu as pltpu
from jax.experimental.pallas import tpu_sc as plsc

NC = 2
NS = 16
NW = NC * NS
CHUNK = 128
D = 128
ROW_BLK = 1280


def _sc_mesh():
    return plsc.VectorSubcoreMesh(
        core_axis_name="c", subcore_axis_name="s", num_cores=NC, num_subcores=NS
    )


def _make_deg_kernel(npad, n_chunks, chunk):
    zper = npad // NS

    @functools.partial(
        pl.kernel,
        out_type=jax.ShapeDtypeStruct((NC * npad,), jnp.float32),
        mesh=_sc_mesh(),
        scratch_types=[
            pltpu.VMEM((n_chunks, chunk), jnp.int32),
            pltpu.VMEM((chunk,), jnp.float32),
            pltpu.VMEM((zper,), jnp.float32),
            pltpu.VMEM_SHARED((npad,), jnp.float32),
        ],
    )
    def deg_kernel(dst_hbm, out_hbm, dst_v, ones_v, zbuf, acc):
        cid = lax.axis_index("c")
        sid = lax.axis_index("s")
        wid = sid * NC + cid

        one16 = jnp.ones((16,), jnp.float32)
        zero16 = jnp.zeros((16,), jnp.float32)

        def fill_ones(i, _):
            ones_v[pl.ds(i * 16, 16)] = one16
            return 0

        lax.fori_loop(0, chunk // 16, fill_ones, 0)

        def fill_zero(i, _):
            zbuf[pl.ds(i * 16, 16)] = zero16
            return 0

        lax.fori_loop(0, zper // 16, fill_zero, 0)

        pltpu.sync_copy(zbuf, acc.at[pl.ds(sid * zper, zper)])
        plsc.subcore_barrier()

        pltpu.sync_copy(dst_hbm.at[wid], dst_v)

        def body(j, _):
            pltpu.sync_copy(ones_v, acc.at[dst_v.at[j]], add=True)
            return 0

        lax.fori_loop(0, n_chunks, body, 0)
        plsc.subcore_barrier()
        pltpu.sync_copy(
            acc.at[pl.ds(sid * zper, zper)],
            out_hbm.at[pl.ds(cid * npad + sid * zper, zper)],
        )

    return deg_kernel


def _make_edge_kernel(npad, n_chunks, chunk, nphase):
    zrows = npad // NS
    assert n_chunks % (2 * nphase) == 0
    hc = n_chunks // nphase
    dp = D // 2

    @functools.partial(
        pl.kernel,
        out_type=jax.ShapeDtypeStruct((NC, npad, D), jnp.float32),
        mesh=_sc_mesh(),
        scratch_types=[
            pltpu.VMEM((hc, chunk), jnp.int32),
            pltpu.VMEM((hc, chunk), jnp.int32),
            pltpu.VMEM((chunk, dp), jnp.int32),
            pltpu.VMEM((chunk, dp), jnp.int32),
            pltpu.VMEM((chunk, D), jnp.float32),
            pltpu.VMEM_SHARED((npad, D), jnp.float32),
            pltpu.SemaphoreType.DMA,
            pltpu.SemaphoreType.DMA,
        ],
        compiler_params=pltpu.CompilerParams(
            use_tc_tiling_on_sc=False, needs_layout_passes=False
        ),
    )
    def edge_kernel(src_hbm, dst_hbm, g_hbm, out_hbm, src_v, dst_v, ibufa,
                    ibufb, fbuf, acc, sema, semb):
        cid = lax.axis_index("c")
        sid = lax.axis_index("s")
        wid = sid * NC + cid

        row0 = sid * zrows
        zero16 = jnp.zeros((16,), jnp.float32)

        def zb(i, _):
            fbuf[i // (D // 16), pl.ds((i % (D // 16)) * 16, 16)] = zero16
            return 0

        lax.fori_loop(0, chunk * (D // 16), zb, 0)

        nfull, rem = divmod(zrows, chunk)
        for k in range(nfull):
            pltpu.sync_copy(fbuf, acc.at[pl.ds(row0 + k * chunk, chunk)])
        if rem:
            pltpu.sync_copy(
                fbuf.at[pl.ds(0, rem)],
                acc.at[pl.ds(row0 + nfull * chunk, rem)],
            )
        plsc.subcore_barrier()

        mask = jnp.full((16,), -65536, jnp.int32)

        def unpack_to_fbuf(ibuf):
            def cv(r, _):
                for c in range(dp // 16):
                    x = ibuf[r, pl.ds(c * 16, 16)]
                    fbuf[r, pl.ds(c * 32, 16)] = plsc.bitcast(
                        lax.shift_left(x, 16), jnp.float32
                    )
                    fbuf[r, pl.ds(c * 32 + 16, 16)] = plsc.bitcast(
                        lax.bitwise_and(x, mask), jnp.float32
                    )
                return 0

            lax.fori_loop(0, chunk, cv, 0)

        for phase in range(nphase):
            pltpu.sync_copy(src_hbm.at[wid, pl.ds(phase * hc, hc)], src_v)
            pltpu.sync_copy(dst_hbm.at[wid, pl.ds(phase * hc, hc)], dst_v)

            pltpu.async_copy(g_hbm.at[src_v.at[0]], ibufa, sema)

            def body(k, _):
                j0 = 2 * k
                j1 = j0 + 1

                pltpu.async_copy(g_hbm.at[src_v.at[j1]], ibufb, semb)
                pltpu.make_async_copy(g_hbm.at[src_v.at[j0]], ibufa, sema).wait()
                unpack_to_fbuf(ibufa)
                pltpu.sync_copy(fbuf, acc.at[dst_v.at[j0]], add=True)

                @pl.when(j0 + 2 < hc)
                def _():
                    pltpu.async_copy(g_hbm.at[src_v.at[j0 + 2]], ibufa, sema)

                pltpu.make_async_copy(g_hbm.at[src_v.at[j1]], ibufb, semb).wait()
                unpack_to_fbuf(ibufb)
                pltpu.sync_copy(fbuf, acc.at[dst_v.at[j1]], add=True)
                return 0

            lax.fori_loop(0, hc // 2, body, 0)

        plsc.subcore_barrier()
        pltpu.sync_copy(
            acc.at[pl.ds(row0, zrows)], out_hbm.at[cid, pl.ds(row0, zrows)]
        )

    return edge_kernel


def _dinv_body(c_ref, o_ref):
    o_ref[...] = lax.rsqrt(c_ref[0] + c_ref[1] + 1.0)


def _pack_g(g):
    ga = jnp.concatenate(
        [g[:, 32 * c:32 * c + 16] for c in range(D // 32)], axis=1
    )
    gb = jnp.concatenate(
        [g[:, 32 * c + 16:32 * c + 32] for c in range(D // 32)], axis=1
    )
    la = lax.bitcast_convert_type(
        ga.astype(jnp.bfloat16), jnp.uint16
    ).astype(jnp.int32)
    hb = lax.bitcast_convert_type(
        gb.astype(jnp.bfloat16), jnp.uint16
    ).astype(jnp.int32)
    return lax.bitwise_or(la, lax.shift_left(hb, 16))


def _mm_scale_body(x_ref, w_ref, s_ref, o_ref, p_ref):
    g = (
        jnp.dot(x_ref[...], w_ref[...], preferred_element_type=jnp.float32)
        * s_ref[...]
    )
    o_ref[...] = g
    p_ref[...] = _pack_g(g)


def _elu(t):
    return jnp.where(t > 0, t, jnp.exp(jnp.minimum(t, 0.0)) - 1.0)


def _layer_mid_body(a_ref, g_ref, s_ref, b_ref, w_ref, o_ref, p_ref):
    s = s_ref[...]
    t = s * (a_ref[0] + a_ref[1] + g_ref[...]) + b_ref[...]
    g = jnp.dot(_elu(t), w_ref[...], preferred_element_type=jnp.float32) * s
    o_ref[...] = g
    p_ref[...] = _pack_g(g)


def _layer_out_body(a_ref, g_ref, s_ref, b_ref, o_ref):
    t = s_ref[...] * (a_ref[0] + a_ref[1] + g_ref[...]) + b_ref[...]
    o_ref[...] = _elu(t)


def _row_grid(npad):
    nb = npad // ROW_BLK
    spec_x = pl.BlockSpec((ROW_BLK, D), lambda i: (i, 0))
    spec_a = pl.BlockSpec((NC, ROW_BLK, D), lambda i: (0, i, 0))
    spec_s = pl.BlockSpec((ROW_BLK, 1), lambda i: (i, 0))
    spec_w = pl.BlockSpec((D, D), lambda i: (0, 0))
    spec_b = pl.BlockSpec((1, D), lambda i: (0, 0))
    return nb, spec_x, spec_a, spec_s, spec_w, spec_b


def kernel(x, edge_index, W1, b1, W2, b2):
    n = x.shape[0]
    e = edge_index.shape[1]
    assert x.shape[1] == D and W1.shape == (D, D)

    npad = -(-(n + 1) // ROW_BLK) * ROW_BLK
    chunk, nphase = 128, 2
    ec = NW * chunk
    grp = 2 * nphase
    n_chunks = ((e + ec - 1) // ec + grp - 1) // grp * grp
    e_pad = n_chunks * ec

    src = edge_index[0]
    dst = edge_index[1]
    src2 = jnp.concatenate(
        [src, jnp.zeros((e_pad - e,), jnp.int32)]
    ).reshape(NW, n_chunks, chunk)
    dst2 = jnp.concatenate(
        [dst, jnp.full((e_pad - e,), n, jnp.int32)]
    ).reshape(NW, n_chunks, chunk)

    x_p = jnp.pad(x, ((0, npad - n), (0, 0)))
    b1r = b1.reshape(1, D)
    b2r = b2.reshape(1, D)

    counts = _make_deg_kernel(npad, n_chunks, chunk)(dst2).reshape(NC, npad)

    dinv2 = pl.pallas_call(
        _dinv_body,
        out_shape=jax.ShapeDtypeStruct((npad // D, D), jnp.float32),
    )(counts.reshape(NC, npad // D, D))
    dinv = dinv2.reshape(npad, 1)

    nb, spec_x, spec_a, spec_s, spec_w, spec_b = _row_grid(npad)

    spec_p = pl.BlockSpec((ROW_BLK, D // 2), lambda i: (i, 0))
    g_shapes = [
        jax.ShapeDtypeStruct((npad, D), jnp.float32),
        jax.ShapeDtypeStruct((npad, D // 2), jnp.int32),
    ]

    g1, g1p = pl.pallas_call(
        _mm_scale_body,
        grid=(nb,),
        in_specs=[spec_x, spec_w, spec_s],
        out_specs=[spec_x, spec_p],
        out_shape=g_shapes,
    )(x_p, W1, dinv)

    edge_kernel = _make_edge_kernel(npad, n_chunks, chunk, nphase)

    acc1 = edge_kernel(src2, dst2, g1p)

    g2, g2p = pl.pallas_call(
        _layer_mid_body,
        grid=(nb,),
        in_specs=[spec_a, spec_x, spec_s, spec_b, spec_w],
        out_specs=[spec_x, spec_p],
        out_shape=g_shapes,
    )(acc1, g1, dinv, b1r, W2)

    acc2 = edge_kernel(src2, dst2, g2p)

    out = pl.pallas_call(
        _layer_out_body,
        grid=(nb,),
        in_specs=[spec_a, spec_x, spec_s, spec_b],
        out_specs=spec_x,
        out_shape=jax.ShapeDtypeStruct((npad, D), jnp.float32),
    )(acc2, g2, dinv, b2r)

    return out[:n]


# speedup vs baseline: 14.6749x; 1.0131x over previous
import functools

import jax
import jax.numpy as jnp
from jax import lax
from jax.experimental import pallas as pl
from jax.experimental.pallas import tpu as pltpu
from jax.experimental.pallas import tpu_sc as plsc

NC = 2
NS = 16
NW = NC * NS
CHUNK = 128
D = 128
ROW_BLK = 1280


def _sc_mesh():
    return plsc.VectorSubcoreMesh(
        core_axis_name="c", subcore_axis_name="s", num_cores=NC, num_subcores=NS
    )


def _make_deg_kernel(npad, n_chunks, chunk):
    zper = npad // NS

    @functools.partial(
        pl.kernel,
        out_type=jax.ShapeDtypeStruct((NC * npad,), jnp.float32),
        mesh=_sc_mesh(),
        scratch_types=[
            pltpu.VMEM((n_chunks, chunk), jnp.int32),
            pltpu.VMEM((chunk,), jnp.float32),
            pltpu.VMEM((zper,), jnp.float32),
            pltpu.VMEM_SHARED((npad,), jnp.float32),
        ],
    )
    def deg_kernel(dst_hbm, out_hbm, dst_v, ones_v, zbuf, acc):
        cid = lax.axis_index("c")
        sid = lax.axis_index("s")
        wid = sid * NC + cid

        one16 = jnp.ones((16,), jnp.float32)
        zero16 = jnp.zeros((16,), jnp.float32)

        def fill_ones(i, _):
            ones_v[pl.ds(i * 16, 16)] = one16
            return 0

        lax.fori_loop(0, chunk // 16, fill_ones, 0)

        def fill_zero(i, _):
            zbuf[pl.ds(i * 16, 16)] = zero16
            return 0

        lax.fori_loop(0, zper // 16, fill_zero, 0)

        pltpu.sync_copy(zbuf, acc.at[pl.ds(sid * zper, zper)])
        plsc.subcore_barrier()

        pltpu.sync_copy(dst_hbm.at[wid], dst_v)

        def body(j, _):
            pltpu.sync_copy(ones_v, acc.at[dst_v.at[j]], add=True)
            return 0

        lax.fori_loop(0, n_chunks, body, 0)
        plsc.subcore_barrier()
        pltpu.sync_copy(
            acc.at[pl.ds(sid * zper, zper)],
            out_hbm.at[pl.ds(cid * npad + sid * zper, zper)],
        )

    return deg_kernel


def _make_edge_kernel(npad, n_chunks, chunk, nphase):
    zrows = npad // NS
    assert n_chunks % (2 * nphase) == 0
    hc = n_chunks // nphase
    dp = D // 2

    @functools.partial(
        pl.kernel,
        out_type=jax.ShapeDtypeStruct((NC, npad, D), jnp.float32),
        mesh=_sc_mesh(),
        scratch_types=[
            pltpu.VMEM((hc, chunk), jnp.int32),
            pltpu.VMEM((hc, chunk), jnp.int32),
            pltpu.VMEM((chunk, dp), jnp.int32),
            pltpu.VMEM((chunk, dp), jnp.int32),
            pltpu.VMEM((chunk, D), jnp.float32),
            pltpu.VMEM_SHARED((npad, D), jnp.float32),
            pltpu.SemaphoreType.DMA,
            pltpu.SemaphoreType.DMA,
            pltpu.SemaphoreType.DMA,
        ],
        compiler_params=pltpu.CompilerParams(
            use_tc_tiling_on_sc=False, needs_layout_passes=False
        ),
    )
    def edge_kernel(src_hbm, dst_hbm, g_hbm, out_hbm, src_v, dst_v, ibufa,
                    ibufb, fbuf, acc, sema, semb, sems):
        cid = lax.axis_index("c")
        sid = lax.axis_index("s")
        wid = sid * NC + cid

        row0 = sid * zrows
        zero16 = jnp.zeros((16,), jnp.float32)

        def zb(i, _):
            fbuf[i // (D // 16), pl.ds((i % (D // 16)) * 16, 16)] = zero16
            return 0

        lax.fori_loop(0, chunk * (D // 16), zb, 0)

        nfull, rem = divmod(zrows, chunk)
        for k in range(nfull):
            pltpu.sync_copy(fbuf, acc.at[pl.ds(row0 + k * chunk, chunk)])
        if rem:
            pltpu.sync_copy(
                fbuf.at[pl.ds(0, rem)],
                acc.at[pl.ds(row0 + nfull * chunk, rem)],
            )
        plsc.subcore_barrier()

        mask = jnp.full((16,), -65536, jnp.int32)

        def unpack_to_fbuf(ibuf):
            def cv(r, _):
                for c in range(dp // 16):
                    x = ibuf[r, pl.ds(c * 16, 16)]
                    fbuf[r, pl.ds(c * 32, 16)] = plsc.bitcast(
                        lax.shift_left(x, 16), jnp.float32
                    )
                    fbuf[r, pl.ds(c * 32 + 16, 16)] = plsc.bitcast(
                        lax.bitwise_and(x, mask), jnp.float32
                    )
                return 0

            lax.fori_loop(0, chunk, cv, 0)

        for phase in range(nphase):
            pltpu.sync_copy(src_hbm.at[wid, pl.ds(phase * hc, hc)], src_v)
            pltpu.sync_copy(dst_hbm.at[wid, pl.ds(phase * hc, hc)], dst_v)

            pltpu.async_copy(g_hbm.at[src_v.at[0]], ibufa, sema)

            def body(k, _):
                j0 = 2 * k
                j1 = j0 + 1

                pltpu.async_copy(g_hbm.at[src_v.at[j1]], ibufb, semb)
                pltpu.make_async_copy(g_hbm.at[src_v.at[j0]], ibufa, sema).wait()

                @pl.when(j0 > 0)
                def _():
                    pltpu.make_async_copy(
                        fbuf, acc.at[dst_v.at[j0 - 1]], sems
                    ).wait()

                unpack_to_fbuf(ibufa)
                pltpu.async_copy(fbuf, acc.at[dst_v.at[j0]], sems, add=True)

                @pl.when(j0 + 2 < hc)
                def _():
                    pltpu.async_copy(g_hbm.at[src_v.at[j0 + 2]], ibufa, sema)

                pltpu.make_async_copy(g_hbm.at[src_v.at[j1]], ibufb, semb).wait()
                pltpu.make_async_copy(fbuf, acc.at[dst_v.at[j0]], sems).wait()
                unpack_to_fbuf(ibufb)
                pltpu.async_copy(fbuf, acc.at[dst_v.at[j1]], sems, add=True)
                return 0

            lax.fori_loop(0, hc // 2, body, 0)
            pltpu.make_async_copy(fbuf, acc.at[dst_v.at[hc - 1]], sems).wait()

        plsc.subcore_barrier()
        pltpu.sync_copy(
            acc.at[pl.ds(row0, zrows)], out_hbm.at[cid, pl.ds(row0, zrows)]
        )

    return edge_kernel


def _dinv_body(c_ref, o_ref):
    o_ref[...] = lax.rsqrt(c_ref[0] + c_ref[1] + 1.0)


def _pack_g(g):
    ga = jnp.concatenate(
        [g[:, 32 * c:32 * c + 16] for c in range(D // 32)], axis=1
    )
    gb = jnp.concatenate(
        [g[:, 32 * c + 16:32 * c + 32] for c in range(D // 32)], axis=1
    )
    la = lax.bitcast_convert_type(
        ga.astype(jnp.bfloat16), jnp.uint16
    ).astype(jnp.int32)
    hb = lax.bitcast_convert_type(
        gb.astype(jnp.bfloat16), jnp.uint16
    ).astype(jnp.int32)
    return lax.bitwise_or(la, lax.shift_left(hb, 16))


def _mm_scale_body(x_ref, w_ref, s_ref, o_ref, p_ref):
    g = (
        jnp.dot(x_ref[...], w_ref[...], preferred_element_type=jnp.float32)
        * s_ref[...]
    )
    o_ref[...] = g
    p_ref[...] = _pack_g(g)


def _elu(t):
    return jnp.where(t > 0, t, jnp.exp(jnp.minimum(t, 0.0)) - 1.0)


def _layer_mid_body(a_ref, g_ref, s_ref, b_ref, w_ref, o_ref, p_ref):
    s = s_ref[...]
    t = s * (a_ref[0] + a_ref[1] + g_ref[...]) + b_ref[...]
    g = jnp.dot(_elu(t), w_ref[...], preferred_element_type=jnp.float32) * s
    o_ref[...] = g
    p_ref[...] = _pack_g(g)


def _layer_out_body(a_ref, g_ref, s_ref, b_ref, o_ref):
    t = s_ref[...] * (a_ref[0] + a_ref[1] + g_ref[...]) + b_ref[...]
    o_ref[...] = _elu(t)


def _row_grid(npad):
    nb = npad // ROW_BLK
    spec_x = pl.BlockSpec((ROW_BLK, D), lambda i: (i, 0))
    spec_a = pl.BlockSpec((NC, ROW_BLK, D), lambda i: (0, i, 0))
    spec_s = pl.BlockSpec((ROW_BLK, 1), lambda i: (i, 0))
    spec_w = pl.BlockSpec((D, D), lambda i: (0, 0))
    spec_b = pl.BlockSpec((1, D), lambda i: (0, 0))
    return nb, spec_x, spec_a, spec_s, spec_w, spec_b


def kernel(x, edge_index, W1, b1, W2, b2):
    n = x.shape[0]
    e = edge_index.shape[1]
    assert x.shape[1] == D and W1.shape == (D, D)

    npad = -(-(n + 1) // ROW_BLK) * ROW_BLK
    chunk, nphase = 128, 2
    ec = NW * chunk
    grp = 2 * nphase
    n_chunks = ((e + ec - 1) // ec + grp - 1) // grp * grp
    e_pad = n_chunks * ec

    src = edge_index[0]
    dst = edge_index[1]
    src2 = jnp.concatenate(
        [src, jnp.zeros((e_pad - e,), jnp.int32)]
    ).reshape(NW, n_chunks, chunk)
    dst2 = jnp.concatenate(
        [dst, jnp.full((e_pad - e,), n, jnp.int32)]
    ).reshape(NW, n_chunks, chunk)

    x_p = jnp.pad(x, ((0, npad - n), (0, 0)))
    b1r = b1.reshape(1, D)
    b2r = b2.reshape(1, D)

    counts = _make_deg_kernel(npad, n_chunks, chunk)(dst2).reshape(NC, npad)

    dinv2 = pl.pallas_call(
        _dinv_body,
        out_shape=jax.ShapeDtypeStruct((npad // D, D), jnp.float32),
    )(counts.reshape(NC, npad // D, D))
    dinv = dinv2.reshape(npad, 1)

    nb, spec_x, spec_a, spec_s, spec_w, spec_b = _row_grid(npad)

    spec_p = pl.BlockSpec((ROW_BLK, D // 2), lambda i: (i, 0))
    g_shapes = [
        jax.ShapeDtypeStruct((npad, D), jnp.float32),
        jax.ShapeDtypeStruct((npad, D // 2), jnp.int32),
    ]

    g1, g1p = pl.pallas_call(
        _mm_scale_body,
        grid=(nb,),
        in_specs=[spec_x, spec_w, spec_s],
        out_specs=[spec_x, spec_p],
        out_shape=g_shapes,
    )(x_p, W1, dinv)

    edge_kernel = _make_edge_kernel(npad, n_chunks, chunk, nphase)

    acc1 = edge_kernel(src2, dst2, g1p)

    g2, g2p = pl.pallas_call(
        _layer_mid_body,
        grid=(nb,),
        in_specs=[spec_a, spec_x, spec_s, spec_b, spec_w],
        out_specs=[spec_x, spec_p],
        out_shape=g_shapes,
    )(acc1, g1, dinv, b1r, W2)

    acc2 = edge_kernel(src2, dst2, g2p)

    out = pl.pallas_call(
        _layer_out_body,
        grid=(nb,),
        in_specs=[spec_a, spec_x, spec_s, spec_b],
        out_specs=spec_x,
        out_shape=jax.ShapeDtypeStruct((npad, D), jnp.float32),
    )(acc2, g2, dinv, b2r)

    return out[:n]


# speedup vs baseline: 14.9949x; 1.0218x over previous
import functools

import jax
import jax.numpy as jnp
from jax import lax
from jax.experimental import pallas as pl
from jax.experimental.pallas import tpu as pltpu
from jax.experimental.pallas import tpu_sc as plsc

NC = 2
NS = 16
NW = NC * NS
CHUNK = 128
D = 128
ROW_BLK = 1280


def _sc_mesh():
    return plsc.VectorSubcoreMesh(
        core_axis_name="c", subcore_axis_name="s", num_cores=NC, num_subcores=NS
    )


def _make_deg_kernel(npad, n_chunks, chunk):
    zper = npad // NS

    @functools.partial(
        pl.kernel,
        out_type=jax.ShapeDtypeStruct((NC * npad,), jnp.float32),
        mesh=_sc_mesh(),
        scratch_types=[
            pltpu.VMEM((n_chunks, chunk), jnp.int32),
            pltpu.VMEM((chunk,), jnp.float32),
            pltpu.VMEM((zper,), jnp.float32),
            pltpu.VMEM_SHARED((npad,), jnp.float32),
        ],
    )
    def deg_kernel(dst_hbm, out_hbm, dst_v, ones_v, zbuf, acc):
        cid = lax.axis_index("c")
        sid = lax.axis_index("s")
        wid = sid * NC + cid

        one16 = jnp.ones((16,), jnp.float32)
        zero16 = jnp.zeros((16,), jnp.float32)

        def fill_ones(i, _):
            ones_v[pl.ds(i * 16, 16)] = one16
            return 0

        lax.fori_loop(0, chunk // 16, fill_ones, 0)

        def fill_zero(i, _):
            zbuf[pl.ds(i * 16, 16)] = zero16
            return 0

        lax.fori_loop(0, zper // 16, fill_zero, 0)

        pltpu.sync_copy(zbuf, acc.at[pl.ds(sid * zper, zper)])
        plsc.subcore_barrier()

        pltpu.sync_copy(dst_hbm.at[wid], dst_v)

        def body(j, _):
            pltpu.sync_copy(ones_v, acc.at[dst_v.at[j]], add=True)
            return 0

        lax.fori_loop(0, n_chunks, body, 0)
        plsc.subcore_barrier()
        pltpu.sync_copy(
            acc.at[pl.ds(sid * zper, zper)],
            out_hbm.at[pl.ds(cid * npad + sid * zper, zper)],
        )

    return deg_kernel


def _make_edge_kernel(npad, n_chunks, chunk, nphase):
    zrows = npad // NS
    assert n_chunks % (2 * nphase) == 0
    hc = n_chunks // nphase
    dp = D // 2

    @functools.partial(
        pl.kernel,
        out_type=jax.ShapeDtypeStruct((NC, npad, D), jnp.float32),
        mesh=_sc_mesh(),
        scratch_types=[
            pltpu.VMEM((hc, chunk), jnp.int32),
            pltpu.VMEM((hc, chunk), jnp.int32),
            pltpu.VMEM((chunk, dp), jnp.int32),
            pltpu.VMEM((chunk, dp), jnp.int32),
            pltpu.VMEM((chunk, D), jnp.float32),
            pltpu.VMEM_SHARED((npad, D), jnp.float32),
            pltpu.SemaphoreType.DMA,
            pltpu.SemaphoreType.DMA,
            pltpu.SemaphoreType.DMA,
        ],
        compiler_params=pltpu.CompilerParams(
            use_tc_tiling_on_sc=False, needs_layout_passes=False
        ),
    )
    def edge_kernel(src_hbm, dst_hbm, g_hbm, out_hbm, src_v, dst_v, ibufa,
                    ibufb, fbuf, acc, sema, semb, sems):
        cid = lax.axis_index("c")
        sid = lax.axis_index("s")
        wid = sid * NC + cid

        row0 = sid * zrows
        zero16 = jnp.zeros((16,), jnp.float32)

        def zb(i, _):
            fbuf[i // (D // 16), pl.ds((i % (D // 16)) * 16, 16)] = zero16
            return 0

        lax.fori_loop(0, chunk * (D // 16), zb, 0)

        nfull, rem = divmod(zrows, chunk)
        for k in range(nfull):
            pltpu.sync_copy(fbuf, acc.at[pl.ds(row0 + k * chunk, chunk)])
        if rem:
            pltpu.sync_copy(
                fbuf.at[pl.ds(0, rem)],
                acc.at[pl.ds(row0 + nfull * chunk, rem)],
            )
        plsc.subcore_barrier()

        mask = jnp.full((16,), -65536, jnp.int32)

        def unpack_to_fbuf(ibuf):
            def cv(r, _):
                for c in range(dp // 16):
                    x = ibuf[r, pl.ds(c * 16, 16)]
                    fbuf[r, pl.ds(c * 32, 16)] = plsc.bitcast(
                        lax.shift_left(x, 16), jnp.float32
                    )
                    fbuf[r, pl.ds(c * 32 + 16, 16)] = plsc.bitcast(
                        lax.bitwise_and(x, mask), jnp.float32
                    )
                return 0

            lax.fori_loop(0, chunk, cv, 0)

        for phase in range(nphase):
            pltpu.sync_copy(src_hbm.at[wid, pl.ds(phase * hc, hc)], src_v)
            pltpu.sync_copy(dst_hbm.at[wid, pl.ds(phase * hc, hc)], dst_v)

            pltpu.async_copy(g_hbm.at[src_v.at[0]], ibufa, sema)

            def body(k, _):
                j0 = 2 * k
                j1 = j0 + 1

                pltpu.async_copy(g_hbm.at[src_v.at[j1]], ibufb, semb)
                pltpu.make_async_copy(g_hbm.at[src_v.at[j0]], ibufa, sema).wait()

                @pl.when(j0 > 0)
                def _():
                    pltpu.make_async_copy(
                        fbuf, acc.at[dst_v.at[j0 - 1]], sems
                    ).wait()

                unpack_to_fbuf(ibufa)
                pltpu.async_copy(fbuf, acc.at[dst_v.at[j0]], sems, add=True)

                @pl.when(j0 + 2 < hc)
                def _():
                    pltpu.async_copy(g_hbm.at[src_v.at[j0 + 2]], ibufa, sema)

                pltpu.make_async_copy(g_hbm.at[src_v.at[j1]], ibufb, semb).wait()
                pltpu.make_async_copy(fbuf, acc.at[dst_v.at[j0]], sems).wait()
                unpack_to_fbuf(ibufb)
                pltpu.async_copy(fbuf, acc.at[dst_v.at[j1]], sems, add=True)
                return 0

            lax.fori_loop(0, hc // 2, body, 0)
            pltpu.make_async_copy(fbuf, acc.at[dst_v.at[hc - 1]], sems).wait()

        plsc.subcore_barrier()
        pltpu.sync_copy(
            acc.at[pl.ds(row0, zrows)], out_hbm.at[cid, pl.ds(row0, zrows)]
        )

    return edge_kernel


def _dinv_body(c_ref, o_ref):
    o_ref[...] = lax.rsqrt(c_ref[0] + c_ref[1] + 1.0)


def _pack_g(g):
    ga = jnp.concatenate(
        [g[:, 32 * c:32 * c + 16] for c in range(D // 32)], axis=1
    )
    gb = jnp.concatenate(
        [g[:, 32 * c + 16:32 * c + 32] for c in range(D // 32)], axis=1
    )
    la = lax.bitcast_convert_type(
        ga.astype(jnp.bfloat16), jnp.uint16
    ).astype(jnp.int32)
    hb = lax.bitcast_convert_type(
        gb.astype(jnp.bfloat16), jnp.uint16
    ).astype(jnp.int32)
    return lax.bitwise_or(la, lax.shift_left(hb, 16))


def _mm_scale_body(x_ref, w_ref, c_ref, o_ref, p_ref, s_ref):
    s = lax.rsqrt(c_ref[0] + c_ref[1] + 1.0)
    g = jnp.dot(x_ref[...], w_ref[...], preferred_element_type=jnp.float32) * s
    o_ref[...] = g
    p_ref[...] = _pack_g(g)
    s_ref[...] = s


def _elu(t):
    return jnp.where(t > 0, t, jnp.exp(jnp.minimum(t, 0.0)) - 1.0)


def _layer_mid_body(a_ref, g_ref, s_ref, b_ref, w_ref, o_ref, p_ref):
    s = s_ref[...]
    t = s * (a_ref[0] + a_ref[1] + g_ref[...]) + b_ref[...]
    g = jnp.dot(_elu(t), w_ref[...], preferred_element_type=jnp.float32) * s
    o_ref[...] = g
    p_ref[...] = _pack_g(g)


def _layer_out_body(a_ref, g_ref, s_ref, b_ref, o_ref):
    t = s_ref[...] * (a_ref[0] + a_ref[1] + g_ref[...]) + b_ref[...]
    o_ref[...] = _elu(t)


def _row_grid(npad):
    nb = npad // ROW_BLK
    spec_x = pl.BlockSpec((ROW_BLK, D), lambda i: (i, 0))
    spec_a = pl.BlockSpec((NC, ROW_BLK, D), lambda i: (0, i, 0))
    spec_s = pl.BlockSpec((ROW_BLK, 1), lambda i: (i, 0))
    spec_w = pl.BlockSpec((D, D), lambda i: (0, 0))
    spec_b = pl.BlockSpec((1, D), lambda i: (0, 0))
    return nb, spec_x, spec_a, spec_s, spec_w, spec_b


def kernel(x, edge_index, W1, b1, W2, b2):
    n = x.shape[0]
    e = edge_index.shape[1]
    assert x.shape[1] == D and W1.shape == (D, D)

    npad = -(-(n + 1) // ROW_BLK) * ROW_BLK
    chunk, nphase = 128, 2
    ec = NW * chunk
    grp = 2 * nphase
    n_chunks = ((e + ec - 1) // ec + grp - 1) // grp * grp
    e_pad = n_chunks * ec

    src = edge_index[0]
    dst = edge_index[1]
    src2 = jnp.concatenate(
        [src, jnp.zeros((e_pad - e,), jnp.int32)]
    ).reshape(NW, n_chunks, chunk)
    dst2 = jnp.concatenate(
        [dst, jnp.full((e_pad - e,), n, jnp.int32)]
    ).reshape(NW, n_chunks, chunk)

    x_p = jnp.pad(x, ((0, npad - n), (0, 0)))
    b1r = b1.reshape(1, D)
    b2r = b2.reshape(1, D)

    counts = _make_deg_kernel(npad, n_chunks, chunk)(dst2).reshape(NC, npad)

    nb, spec_x, spec_a, spec_s, spec_w, spec_b = _row_grid(npad)

    spec_p = pl.BlockSpec((ROW_BLK, D // 2), lambda i: (i, 0))
    spec_c = pl.BlockSpec((NC, ROW_BLK, 1), lambda i: (0, i, 0))
    g_shapes = [
        jax.ShapeDtypeStruct((npad, D), jnp.float32),
        jax.ShapeDtypeStruct((npad, D // 2), jnp.int32),
    ]

    g1, g1p, dinv = pl.pallas_call(
        _mm_scale_body,
        grid=(nb,),
        in_specs=[spec_x, spec_w, spec_c],
        out_specs=[spec_x, spec_p, spec_s],
        out_shape=g_shapes
        + [jax.ShapeDtypeStruct((npad, 1), jnp.float32)],
    )(x_p, W1, counts.reshape(NC, npad, 1))

    edge_kernel = _make_edge_kernel(npad, n_chunks, chunk, nphase)

    acc1 = edge_kernel(src2, dst2, g1p)

    g2, g2p = pl.pallas_call(
        _layer_mid_body,
        grid=(nb,),
        in_specs=[spec_a, spec_x, spec_s, spec_b, spec_w],
        out_specs=[spec_x, spec_p],
        out_shape=g_shapes,
    )(acc1, g1, dinv, b1r, W2)

    acc2 = edge_kernel(src2, dst2, g2p)

    out = pl.pallas_call(
        _layer_out_body,
        grid=(nb,),
        in_specs=[spec_a, spec_x, spec_s, spec_b],
        out_specs=spec_x,
        out_shape=jax.ShapeDtypeStruct((npad, D), jnp.float32),
    )(acc2, g2, dinv, b2r)

    return out[:n]
